# baseline (speedup 1.0000x reference)
import functools

import jax
import jax.numpy as jnp
from jax import lax
from jax.experimental import pallas as pl
from jax.experimental.pallas import tpu as pltpu
from jax.experimental.pallas import tpu_sc as plsc

SEQ = 2048
D = 128
B = 16
NC = 2
NS = 16
NW = NC * NS
LBLK = SEQ // NW
KV = D // 16


def _emb_body(tokens_hbm, types_hbm, pos_hbm, tok_tbl_hbm, typ_tbl_hbm,
              out_hbm, tok_idx, typ_idx, pos_v, typ_v,
              buf0, buf1, obuf0, obuf1,
              ssem, gsem0, gsem1, osem0, osem1):
    cid = lax.axis_index("c")
    sid = lax.axis_index("s")
    wid = sid * NC + cid
    l0 = wid * LBLK
    l0a = (wid // 2) * 128
    coff = (wid % 2) * LBLK

    c1 = pltpu.async_copy(tokens_hbm.at[:, pl.ds(l0a, 128)], tok_idx, ssem)
    c2 = pltpu.async_copy(types_hbm.at[:, pl.ds(l0a, 128)], typ_idx, ssem)
    c3 = pltpu.async_copy(pos_hbm.at[pl.ds(l0, LBLK)], pos_v, ssem)
    c4 = pltpu.async_copy(typ_tbl_hbm, typ_v, ssem)
    c1.wait()

    def gather(b, buf, gsem):
        pltpu.async_copy(
            tok_tbl_hbm.at[tok_idx.at[b, pl.ds(coff, LBLK)]], buf, gsem
        )

    def wait_gather(b, buf, gsem):
        pltpu.make_async_copy(
            tok_tbl_hbm.at[tok_idx.at[b, pl.ds(coff, LBLK)]], buf, gsem
        ).wait()

    gather(0, buf0, gsem0)
    c2.wait()
    c3.wait()
    c4.wait()

    def fold_row(r, carry):
        for k in range(KV):
            s = pl.ds(k * 16, 16)
            pos_v[r, s] = pos_v[r, s] + typ_v[0, s]
        return carry

    lax.fori_loop(0, LBLK, fold_row, 0)

    dvec = [
        typ_v[1, pl.ds(k * 16, 16)] - typ_v[0, pl.ds(k * 16, 16)]
        for k in range(KV)
    ]

    def add_batch(b, buf, obuf):
        def add_group(g, inner):
            base = g * 16
            tvf = typ_idx[b, pl.ds(coff + base, 16)].astype(jnp.float32)
            for jj in range(16):
                t = jnp.broadcast_to(tvf[jj], (16,))
                r = base + jj
                for k in range(KV):
                    s = pl.ds(k * 16, 16)
                    obuf[r, s] = buf[r, s] + pos_v[r, s] + t * dvec[k]
            return inner

        lax.fori_loop(0, LBLK // 16, add_group, 0)

    def half(i, b, buf, obuf, gsem, osem):
        wait_gather(b, buf, gsem)

        @pl.when(i > 0)
        def _():
            pltpu.make_async_copy(
                obuf, out_hbm.at[pl.ds(b * SEQ + l0, LBLK)], osem
            ).wait()

        add_batch(b, buf, obuf)
        pltpu.async_copy(obuf, out_hbm.at[pl.ds(b * SEQ + l0, LBLK)], osem)

    def pair_body(i, carry):
        b0 = 2 * i
        b1 = b0 + 1
        gather(b1, buf1, gsem1)
        half(i, b0, buf0, obuf0, gsem0, osem0)

        @pl.when(i < B // 2 - 1)
        def _():
            gather(b0 + 2, buf0, gsem0)

        half(i, b1, buf1, obuf1, gsem1, osem1)
        return carry

    lax.fori_loop(0, B // 2, pair_body, 0)

    pltpu.make_async_copy(obuf0, out_hbm.at[pl.ds(l0, LBLK)], osem0).wait()
    pltpu.make_async_copy(obuf1, out_hbm.at[pl.ds(l0, LBLK)], osem1).wait()


def kernel(tokens, types, pos_table, tok_table, type_table):
    mesh = plsc.VectorSubcoreMesh(
        core_axis_name="c", subcore_axis_name="s", num_cores=NC, num_subcores=NS
    )
    run = functools.partial(
        pl.kernel,
        mesh=mesh,
        out_type=jax.ShapeDtypeStruct((B * SEQ, D), jnp.float32),
        scratch_types=[
            pltpu.VMEM((B, 128), jnp.int32),
            pltpu.VMEM((B, 128), jnp.int32),
            pltpu.VMEM((LBLK, D), jnp.float32),
            pltpu.VMEM((2, D), jnp.float32),
            pltpu.VMEM((LBLK, D), jnp.float32),
            pltpu.VMEM((LBLK, D), jnp.float32),
            pltpu.VMEM((LBLK, D), jnp.float32),
            pltpu.VMEM((LBLK, D), jnp.float32),
            pltpu.SemaphoreType.DMA,
            pltpu.SemaphoreType.DMA,
            pltpu.SemaphoreType.DMA,
            pltpu.SemaphoreType.DMA,
            pltpu.SemaphoreType.DMA,
        ],
    )(_emb_body)
    out = run(tokens, types, pos_table, tok_table, type_table)
    return out.reshape(B, SEQ, D)


# speedup vs baseline: 2.0054x; 2.0054x over previous
import functools

import jax
import jax.numpy as jnp
from jax import lax
from jax.experimental import pallas as pl
from jax.experimental.pallas import tpu as pltpu
from jax.experimental.pallas import tpu_sc as plsc

SEQ = 2048
D = 128
B = 16
NC = 2
NS = 16
NW = NC * NS
LBLK = SEQ // NW
KV = D // 16
DEPTH = 4


def _emb_body(tokens_hbm, types_hbm, pos_hbm, tok_tbl_hbm, typ_tbl_hbm,
              out_hbm, fused_hbm, tok_idx, typ_idx, pos_v, typ_v, fused_v,
              idxs, bufs, obufs, ssem, gsems, osems):
    cid = lax.axis_index("c")
    sid = lax.axis_index("s")
    wid = sid * NC + cid
    l0 = wid * LBLK
    l0a = (wid // 2) * 128
    coff = (wid % 2) * LBLK
    sbase = wid * 2 * LBLK

    c1 = pltpu.async_copy(tokens_hbm.at[:, pl.ds(l0a, 128)], tok_idx, ssem)
    c2 = pltpu.async_copy(types_hbm.at[:, pl.ds(l0a, 128)], typ_idx, ssem)
    c3 = pltpu.async_copy(pos_hbm.at[pl.ds(l0, LBLK)], pos_v, ssem)
    c4 = pltpu.async_copy(typ_tbl_hbm, typ_v, ssem)
    c1.wait()

    def tok_gather(b, buf, gsem):
        pltpu.async_copy(
            tok_tbl_hbm.at[tok_idx.at[b, pl.ds(coff, LBLK)]], buf, gsem
        )

    for q in range(DEPTH - 1):
        tok_gather(q, bufs[q], gsems[q])

    c2.wait()
    c3.wait()
    c4.wait()

    iota = lax.iota(jnp.int32, 16)
    tv = [typ_v[t, pl.ds(k * 16, 16)] for t in range(2) for k in range(KV)]

    def fuse_row(r, carry):
        for t in range(2):
            for k in range(KV):
                s = pl.ds(k * 16, 16)
                fused_v[t * LBLK + r, s] = pos_v[r, s] + tv[t * KV + k]
        return carry

    lax.fori_loop(0, LBLK, fuse_row, 0)
    pltpu.sync_copy(fused_v, fused_hbm.at[pl.ds(sbase, 2 * LBLK)])

    def fused_gather(b, idx, fbuf, gsem):
        for g in range(LBLK // 16):
            tvec = typ_idx[b, pl.ds(coff + g * 16, 16)]
            idx[pl.ds(g * 16, 16)] = (sbase + g * 16) + iota + tvec * LBLK
        pltpu.async_copy(fused_hbm.at[idx], fbuf, gsem)

    for q in range(DEPTH - 1):
        fused_gather(q, idxs[q], obufs[q], gsems[q])

    def wait_gathers(b, idx, buf, fbuf, gsem):
        pltpu.make_async_copy(
            tok_tbl_hbm.at[tok_idx.at[b, pl.ds(coff, LBLK)]], buf, gsem
        ).wait()
        pltpu.make_async_copy(fused_hbm.at[idx], fbuf, gsem).wait()

    def add_batch(buf, fbuf):
        def add_row(r, carry):
            for k in range(KV):
                s = pl.ds(k * 16, 16)
                fbuf[r, s] = buf[r, s] + fbuf[r, s]
            return carry

        lax.fori_loop(0, LBLK, add_row, 0)

    def quarter(i, q):
        b = DEPTH * i + q
        nq = (q + DEPTH - 1) % DEPTH

        @pl.when(b + DEPTH - 1 < B)
        def _():
            bn = b + DEPTH - 1

            def drain_prev():
                pltpu.make_async_copy(
                    obufs[nq],
                    out_hbm.at[pl.ds((bn - DEPTH) * SEQ + l0, LBLK)],
                    osems[nq],
                ).wait()

            if q == 0:
                pl.when(i > 0)(drain_prev)
            else:
                drain_prev()

            tok_gather(bn, bufs[nq], gsems[nq])
            fused_gather(bn, idxs[nq], obufs[nq], gsems[nq])

        wait_gathers(b, idxs[q], bufs[q], obufs[q], gsems[q])
        add_batch(bufs[q], obufs[q])
        pltpu.async_copy(obufs[q], out_hbm.at[pl.ds(b * SEQ + l0, LBLK)],
                         osems[q])

    def group_body(i, carry):
        for q in range(DEPTH):
            quarter(i, q)
        return carry

    lax.fori_loop(0, B // DEPTH, group_body, 0)

    for q in range(DEPTH):
        pltpu.make_async_copy(
            obufs[q], out_hbm.at[pl.ds(l0, LBLK)], osems[q]
        ).wait()


def kernel(tokens, types, pos_table, tok_table, type_table):
    mesh = plsc.VectorSubcoreMesh(
        core_axis_name="c", subcore_axis_name="s", num_cores=NC, num_subcores=NS
    )
    run = functools.partial(
        pl.kernel,
        mesh=mesh,
        out_type=(
            jax.ShapeDtypeStruct((B * SEQ, D), jnp.float32),
            jax.ShapeDtypeStruct((NW * 2 * LBLK, D), jnp.float32),
        ),
        scratch_types=[
            pltpu.VMEM((B, 128), jnp.int32),
            pltpu.VMEM((B, 128), jnp.int32),
            pltpu.VMEM((LBLK, D), jnp.float32),
            pltpu.VMEM((2, D), jnp.float32),
            pltpu.VMEM((2 * LBLK, D), jnp.float32),
            [pltpu.VMEM((LBLK,), jnp.int32) for _ in range(DEPTH)],
            [pltpu.VMEM((LBLK, D), jnp.float32) for _ in range(DEPTH)],
            [pltpu.VMEM((LBLK, D), jnp.float32) for _ in range(DEPTH)],
            pltpu.SemaphoreType.DMA,
            [pltpu.SemaphoreType.DMA for _ in range(DEPTH)],
            [pltpu.SemaphoreType.DMA for _ in range(DEPTH)],
        ],
    )(_emb_body)
    out, _ = run(tokens, types, pos_table, tok_table, type_table)
    return out.reshape(B, SEQ, D)
